# Initial kernel scaffold; baseline (speedup 1.0000x reference)
import jax
import jax.numpy as jnp
from jax.experimental import pallas as pl


def kernel(probs):
    raise NotImplementedError("write your pallas kernel here")


# speedup vs baseline: 1.4893x; 1.4893x over previous
import jax
import jax.numpy as jnp
from jax.experimental import pallas as pl
from jax.experimental.pallas import tpu as pltpu

B = 128
V = 100000
NBLK = 784
VP = NBLK * 128
TILES = 49
TBLK = 16
VREGS_PER_TILE = 256
CHUNK = 962
OHW = 2048


def _halving(acc):
    a = acc[0:4, :] + acc[4:8, :]
    b = a[0:2, :] + a[2:4, :]
    return b[0:1, :] + b[1:2, :]


def _main_kernel(in_ref, u_ref, ix_ref,
                 acc8, ssum, rcp, l2run, ex3, cexn, cnt, lscr):
    i = pl.program_id(0)

    @pl.when(i == 0)
    def _init():
        acc8[...] = jnp.zeros((8, B), jnp.float32)
        ssum[...] = jnp.zeros((1, B), jnp.float32)
        l2run[...] = jnp.zeros((1, B), jnp.float32)
        ex3[...] = jnp.zeros((1, B), jnp.float32)
        cexn[...] = jnp.zeros((1, B), jnp.float32)
        cnt[...] = jnp.zeros((1, B), jnp.int32)

    @pl.when(i < TILES)
    def _sum_sweep():
        base = i * VREGS_PER_TILE

        def fold(lo, hi, acc):
            def body(k, a):
                jo = jax.lax.rem(k, TBLK)
                bb = jax.lax.div(k, TBLK)
                v = in_ref[pl.ds(jo * 8, 8), bb, :].reshape(8, B)
                return a + v
            return jax.lax.fori_loop(lo, hi, body, acc)

        kstar = jax.lax.rem(CHUNK - jax.lax.rem(base, CHUNK), CHUNK)
        kstar = jnp.where(i == 0, CHUNK, kstar)
        kk = jnp.minimum(kstar, VREGS_PER_TILE)
        acc = fold(0, kk, acc8[...])
        did = kstar < VREGS_PER_TILE
        part = _halving(acc)
        ssum[...] = jnp.where(did, ssum[...] + part, ssum[...])
        acc = jnp.where(did, jnp.zeros_like(acc), acc)
        acc8[...] = fold(kk, VREGS_PER_TILE, acc)

        @pl.when(i == TILES - 1)
        def _finalize():
            rcp[...] = jnp.float32(1.0) / ssum[...]

    @pl.when(i >= TILES)
    def _scan_sweep():
        j_tile = i - TILES
        r = rcp[...]

        def body(jj, runs):
            v = in_ref[jj, :, :].reshape(TBLK, B)
            runs = runs + v * r
            lscr[jj, :, :] = runs
            return runs

        runs0 = jnp.zeros((TBLK, B), jnp.float32)
        bsums = jax.lax.fori_loop(0, 128, body, runs0)

        l2 = l2run[...]
        e3 = ex3[...]
        cx = cexn[...]
        g0 = jax.lax.rem(j_tile, 8) == 0
        e3 = jnp.where(g0, e3 + l2, e3)
        l2 = jnp.where(g0, jnp.zeros_like(l2), l2)
        cex_rows = []
        for bb in range(TBLK):
            l2 = l2 + bsums[bb:bb + 1, :]
            cincl = l2 + e3
            cex_rows.append(cx)
            cx = cincl
        l2run[...] = l2
        ex3[...] = e3
        cexn[...] = cx
        cex16 = jnp.concatenate(cex_rows, axis=0)

        u = u_ref[...]
        tile_v0 = j_tile * (TBLK * 128)
        total = jnp.zeros((1, B), jnp.int32)
        for jc in range(8):
            lpart = lscr[jc * 16:(jc + 1) * 16, :, :]
            cdf = lpart + cex16[None, :, :]
            jj = jax.lax.broadcasted_iota(jnp.int32, (16, TBLK, B), 0)
            bb = jax.lax.broadcasted_iota(jnp.int32, (16, TBLK, B), 1)
            vglob = tile_v0 + bb * 128 + (jc * 16 + jj)
            pred = (cdf < u[None, :, :]) & (vglob < V)
            c = jnp.sum(pred.astype(jnp.int32), axis=(0, 1))
            total = total + c.reshape(1, B)
        cnt[...] = cnt[...] + total

        @pl.when(i == 2 * TILES - 1)
        def _emit():
            ix_ref[...] = jnp.clip(cnt[...], 0, V - 1)


def _onehot_kernel(ix_ref, out_ref):
    i = pl.program_id(0)
    col = jax.lax.broadcasted_iota(jnp.int32, (B, OHW), 1) + i * OHW
    out_ref[...] = (col == ix_ref[...]).astype(jnp.float32)


def kernel(probs):
    u = jax.random.uniform(jax.random.key(42), (B, 1), dtype=probs.dtype)
    u_lanes = u.reshape(1, B)

    ppad = jnp.pad(probs, ((0, 0), (0, VP - V)))
    ptj = jnp.transpose(ppad.reshape(B, NBLK, 128), (2, 1, 0))

    ix = pl.pallas_call(
        _main_kernel,
        grid=(2 * TILES,),
        in_specs=[
            pl.BlockSpec((128, TBLK, B), lambda i: (0, i % TILES, 0)),
            pl.BlockSpec((1, B), lambda i: (0, 0)),
        ],
        out_specs=pl.BlockSpec((1, B), lambda i: (0, 0)),
        out_shape=jax.ShapeDtypeStruct((1, B), jnp.int32),
        scratch_shapes=[
            pltpu.VMEM((8, B), jnp.float32),
            pltpu.VMEM((1, B), jnp.float32),
            pltpu.VMEM((1, B), jnp.float32),
            pltpu.VMEM((1, B), jnp.float32),
            pltpu.VMEM((1, B), jnp.float32),
            pltpu.VMEM((1, B), jnp.float32),
            pltpu.VMEM((1, B), jnp.int32),
            pltpu.VMEM((128, TBLK, B), jnp.float32),
        ],
    )(ptj, u_lanes)

    ixn = ix.reshape(B, 1)
    onehot = pl.pallas_call(
        _onehot_kernel,
        grid=(TILES,),
        in_specs=[pl.BlockSpec((B, 1), lambda i: (0, 0))],
        out_specs=pl.BlockSpec((B, OHW), lambda i: (0, i)),
        out_shape=jax.ShapeDtypeStruct((B, V), jnp.float32),
    )(ixn)
    return onehot
